# baseline (speedup 1.0000x reference)
import functools
import math

import numpy as np
import jax
import jax.numpy as jnp
from jax import lax
from jax.experimental import pallas as pl
from jax.experimental.pallas import tpu as pltpu
from jax.experimental.pallas import tpu_sc as plsc

_D = 64
_L = 50
_B = 4096
_ROWS = _B * _L
_NC = 2
_NS = 16
_NW = _NC * _NS
_RPW = _ROWS // _NW
_SUB = 80
_SPW = _RPW // _SUB
_CHUNK = 16 * _L
_NCHUNK = _RPW // _CHUNK
_SUBS_PER_CHUNK = _CHUNK // _SUB
_LANES = 16
_VPR = _D // _LANES


def _pos_encoding():
    pe = np.zeros((_L, _D), dtype=np.float32)
    pos = np.arange(_L, dtype=np.float32)[:, None]
    div = np.exp(np.arange(0, _D, 2, dtype=np.float32) * -(math.log(10000.0) / _D))
    pe[:, 0::2] = np.sin(pos * div)
    pe[:, 1::2] = np.cos(pos * div)
    return jnp.asarray(pe)


def _emb_body(x_hbm, table_hbm, pe_hbm, out_hbm, idx_v, buf0, buf1, pe_v, gsem):
    wid = lax.axis_index("s") * _NC + lax.axis_index("c")
    sub0 = wid * _SPW
    row0 = wid * _RPW

    pltpu.sync_copy(x_hbm.at[pl.ds(sub0, _SPW), :], idx_v)
    pltpu.sync_copy(pe_hbm, pe_v)

    bufs = (buf0, buf1)

    def issue(c, buf):
        descs = []
        for j in range(_SUBS_PER_CHUNK):
            descs.append(pltpu.async_copy(
                table_hbm.at[idx_v.at[c * _SUBS_PER_CHUNK + j]],
                buf.at[pl.ds(j * _SUB, _SUB), :],
                gsem))
        return descs

    pending = issue(0, bufs[0])
    for c in range(_NCHUNK):
        buf = bufs[c % 2]
        for d in pending:
            d.wait()
        if c + 1 < _NCHUNK:
            pending = issue(c + 1, bufs[(c + 1) % 2])

        @pl.loop(0, _L)
        def _add_pe(l, buf=buf):
            for k in range(_VPR):
                pev = pe_v[l, pl.ds(k * _LANES, _LANES)]
                for s in range(_CHUNK // _L):
                    plsc.addupdate(
                        buf.at[s * _L + l, pl.ds(k * _LANES, _LANES)], pev)

        pltpu.sync_copy(buf, out_hbm.at[pl.ds(row0 + c * _CHUNK, _CHUNK), :])


@functools.lru_cache(maxsize=1)
def _build():
    mesh = plsc.VectorSubcoreMesh(
        core_axis_name="c", subcore_axis_name="s",
        num_cores=_NC, num_subcores=_NS)
    return functools.partial(
        pl.kernel,
        out_type=jax.ShapeDtypeStruct((_ROWS, _D), jnp.float32),
        mesh=mesh,
        scratch_types=[
            pltpu.VMEM((_SPW, _SUB), jnp.int32),
            pltpu.VMEM((_CHUNK, _D), jnp.float32),
            pltpu.VMEM((_CHUNK, _D), jnp.float32),
            pltpu.VMEM((_L, _D), jnp.float32),
            pltpu.SemaphoreType.DMA,
        ],
        compiler_params=pltpu.CompilerParams(use_tc_tiling_on_sc=False),
    )(_emb_body)


def kernel(x, table):
    xf = x.reshape(_ROWS).astype(jnp.int32).reshape(_NW * _SPW, _SUB)
    out = _build()(xf, table, _pos_encoding())
    return out.reshape(_B, _L, _D)
